# Initial kernel scaffold; baseline (speedup 1.0000x reference)
import jax
import jax.numpy as jnp
from jax.experimental import pallas as pl


def kernel(entity, edge_index, edge_type, emb_table, comp1, basis1, root1, bias1, comp2, basis2, root2, bias2):
    raise NotImplementedError("write your pallas kernel here")


# speedup vs baseline: 1.9643x; 1.9643x over previous
import functools

import jax
import jax.numpy as jnp
from jax import lax
from jax.experimental import pallas as pl
from jax.experimental.pallas import tpu as pltpu
from jax.experimental.pallas import tpu_sc as plsc

N = 10000
E = 160000
D = 100
R = 20
NB = 4

NC = 2
NS = 16
NW = NC * NS

NP = 10240
ROWS_W = NP // NW
LCH = 64
EP = 163840
EW = EP // NW
EK = 128
NCHUNK = EW // EK
NRP = 200704
SEGW = NRP // NW
DP = 128
TR = R + 1
DUMP = N
AGG_W = NP // NS

_mesh = plsc.VectorSubcoreMesh(core_axis_name="c", subcore_axis_name="s")


def _sc_lookup_counts_body(entity_ref, emb_ref, dst_ref, t_ref,
                           x_out, inv_out,
                           idx_v, rows_v, ebuf_d, ebuf_t, cnt_v, sem):
    c = lax.axis_index("c")
    s = lax.axis_index("s")
    w = s * NC + c

    pltpu.sync_copy(entity_ref.at[pl.ds(w * ROWS_W, ROWS_W)], idx_v)
    for k in range(ROWS_W // LCH):
        pltpu.async_copy(emb_ref.at[idx_v.at[pl.ds(k * LCH, LCH)]],
                         rows_v.at[pl.ds(k * LCH, LCH)], sem).wait()
    pltpu.sync_copy(rows_v, x_out.at[pl.ds(w * ROWS_W, ROWS_W)])

    def zero_body(i, _):
        cnt_v[pl.ds(i * 16, 16)] = jnp.zeros((16,), jnp.float32)
        return 0
    lax.fori_loop(0, SEGW // 16, zero_body, 0, unroll=4)

    lo = w * SEGW
    ones16 = jnp.ones((16,), jnp.float32)

    def chunk_body(ci, _):
        pltpu.sync_copy(dst_ref.at[pl.ds(ci * EW, EW)], ebuf_d)
        pltpu.sync_copy(t_ref.at[pl.ds(ci * EW, EW)], ebuf_t)

        def vec_body(vi, _):
            d16 = ebuf_d[pl.ds(vi * 16, 16)]
            t16 = ebuf_t[pl.ds(vi * 16, 16)]
            rel = d16 * R + t16 - lo
            m = (rel >= 0) & (rel < SEGW)
            relc = jnp.where(m, rel, 0)
            plsc.addupdate_scatter(cnt_v, [relc], ones16, mask=m)
            return 0
        lax.fori_loop(0, EW // 16, vec_body, 0, unroll=4)
        return 0
    lax.fori_loop(0, EP // EW, chunk_body, 0)

    def inv_body(i, _):
        v = cnt_v[pl.ds(i * 16, 16)]
        cnt_v[pl.ds(i * 16, 16)] = 1.0 / jnp.maximum(v, 1.0)
        return 0
    lax.fori_loop(0, SEGW // 16, inv_body, 0, unroll=4)
    pltpu.sync_copy(cnt_v, inv_out.at[pl.ds(lo, SEGW)])


_sc_lookup_counts = pl.kernel(
    _sc_lookup_counts_body,
    out_type=(jax.ShapeDtypeStruct((NP, DP), jnp.float32),
              jax.ShapeDtypeStruct((NRP,), jnp.float32)),
    mesh=_mesh,
    scratch_types=[
        pltpu.VMEM((ROWS_W,), jnp.int32),
        pltpu.VMEM((ROWS_W, DP), jnp.float32),
        pltpu.VMEM((EW,), jnp.int32),
        pltpu.VMEM((EW,), jnp.int32),
        pltpu.VMEM((SEGW,), jnp.float32),
        pltpu.SemaphoreType.DMA,
    ],
    compiler_params=pltpu.CompilerParams(needs_layout_passes=False, use_tc_tiling_on_sc=False),
)


def _sc_edge_agg_body(xw_ref, inv_ref, zeros_ref, src_ref, dst_ref, t_ref,
                      out_ref,
                      ebuf_s, ebuf_d, ebuf_t, gidx, sidx, didx,
                      rows_v, s_v, agg_sh, sem, sem2):
    c = lax.axis_index("c")
    s = lax.axis_index("s")
    w = s * NC + c

    pltpu.sync_copy(zeros_ref, agg_sh.at[pl.ds(s * AGG_W, AGG_W)])
    plsc.subcore_barrier()

    pltpu.sync_copy(src_ref.at[pl.ds(w * EW, EW)], ebuf_s)
    pltpu.sync_copy(dst_ref.at[pl.ds(w * EW, EW)], ebuf_d)
    pltpu.sync_copy(t_ref.at[pl.ds(w * EW, EW)], ebuf_t)

    def chunk_body(ci, _):
        def bidx(vi, _):
            s16 = ebuf_s[pl.ds(ci * EK + vi * 16, 16)]
            d16 = ebuf_d[pl.ds(ci * EK + vi * 16, 16)]
            t16 = ebuf_t[pl.ds(ci * EK + vi * 16, 16)]
            gidx[pl.ds(vi * 16, 16)] = t16 * NP + s16
            sidx[pl.ds(vi * 16, 16)] = d16 * R + t16
            didx[pl.ds(vi * 16, 16)] = d16
            return 0
        lax.fori_loop(0, EK // 16, bidx, 0, unroll=8)

        cp_rows = pltpu.async_copy(xw_ref.at[gidx], rows_v, sem)
        cp_s = pltpu.async_copy(inv_ref.at[sidx], s_v, sem2)
        cp_rows.wait()
        cp_s.wait()

        def scale_row(r, _):
            sc = plsc.load_gather(s_v, [jnp.full((16,), r, jnp.int32)])
            for g in range(DP // 16):
                v = rows_v[r, pl.ds(g * 16, 16)]
                rows_v[r, pl.ds(g * 16, 16)] = v * sc
            return 0
        lax.fori_loop(0, EK, scale_row, 0)

        pltpu.sync_copy(rows_v, agg_sh.at[didx], add=True)
        return 0
    lax.fori_loop(0, NCHUNK, chunk_body, 0)

    plsc.subcore_barrier()
    pltpu.sync_copy(agg_sh.at[pl.ds(s * AGG_W, AGG_W)],
                    out_ref.at[c, pl.ds(s * AGG_W, AGG_W)])


_sc_edge_agg = pl.kernel(
    _sc_edge_agg_body,
    out_type=jax.ShapeDtypeStruct((NC, NP, DP), jnp.float32),
    mesh=_mesh,
    scratch_types=[
        pltpu.VMEM((EW,), jnp.int32),
        pltpu.VMEM((EW,), jnp.int32),
        pltpu.VMEM((EW,), jnp.int32),
        pltpu.VMEM((EK,), jnp.int32),
        pltpu.VMEM((EK,), jnp.int32),
        pltpu.VMEM((EK,), jnp.int32),
        pltpu.VMEM((EK, DP), jnp.float32),
        pltpu.VMEM((EK,), jnp.float32),
        pltpu.MemorySpace.VMEM_SHARED((NP, DP), jnp.float32),
        pltpu.SemaphoreType.DMA,
        pltpu.SemaphoreType.DMA,
    ],
    compiler_params=pltpu.CompilerParams(needs_layout_passes=False, use_tc_tiling_on_sc=False),
)


def _wbuild_body(comp_ref, basis_ref, root_ref, w_out):
    basis2d = jnp.concatenate(
        [basis_ref[...], jnp.zeros((NB, D, DP - D), jnp.float32)], axis=2
    ).reshape(NB, D * DP)
    w = lax.dot_general(comp_ref[...], basis2d, (((1,), (0,)), ((), ())),
                        preferred_element_type=jnp.float32)
    rootp = jnp.concatenate(
        [root_ref[...], jnp.zeros((D, DP - D), jnp.float32)], axis=1
    ).reshape(1, D * DP)
    w_out[...] = jnp.concatenate([w, rootp], axis=0).reshape(TR, D, DP)


def _wbuild(comp, basis, root):
    return pl.pallas_call(
        _wbuild_body,
        out_shape=jax.ShapeDtypeStruct((TR, D, DP), jnp.float32),
    )(comp, basis, root)


BN = 1024


def _xw_body(x_ref, w_ref, out_ref):
    out_ref[...] = lax.dot_general(
        x_ref[...], w_ref[0], (((1,), (0,)), ((), ())),
        preferred_element_type=jnp.float32)[None, :, :]


def _xw_matmul(x, w):
    k = x.shape[1]
    return pl.pallas_call(
        _xw_body,
        grid=(NP // BN, TR),
        in_specs=[
            pl.BlockSpec((BN, k), lambda i, j: (i, 0)),
            pl.BlockSpec((1, k, DP), lambda i, j: (j, 0, 0)),
        ],
        out_specs=pl.BlockSpec((1, BN, DP), lambda i, j: (j, i, 0)),
        out_shape=jax.ShapeDtypeStruct((TR, NP, DP), jnp.float32),
    )(x, w)


def _combine_matmul_body(agg_ref, self_ref, bias_ref, w_ref, out_ref):
    h = agg_ref[0] + agg_ref[1] + self_ref[...] + bias_ref[...]
    h = jnp.maximum(h, 0.0)
    out_ref[...] = lax.dot_general(
        h, w_ref[0], (((1,), (0,)), ((), ())),
        preferred_element_type=jnp.float32)[None, :, :]


def _combine_matmul(agg, selfloop, bias_p, w):
    return pl.pallas_call(
        _combine_matmul_body,
        grid=(NP // BN, TR),
        in_specs=[
            pl.BlockSpec((NC, BN, DP), lambda i, j: (0, i, 0)),
            pl.BlockSpec((BN, DP), lambda i, j: (i, 0)),
            pl.BlockSpec((1, DP), lambda i, j: (0, 0)),
            pl.BlockSpec((1, DP, DP), lambda i, j: (j, 0, 0)),
        ],
        out_specs=pl.BlockSpec((1, BN, DP), lambda i, j: (j, i, 0)),
        out_shape=jax.ShapeDtypeStruct((TR, NP, DP), jnp.float32),
    )(agg, selfloop, bias_p, w)


def _combine_final_body(agg_ref, self_ref, bias_ref, out_ref):
    out_ref[...] = agg_ref[0] + agg_ref[1] + self_ref[...] + bias_ref[...]


def _combine_final(agg, selfloop, bias_p):
    return pl.pallas_call(
        _combine_final_body,
        grid=(NP // BN,),
        in_specs=[
            pl.BlockSpec((NC, BN, DP), lambda i: (0, i, 0)),
            pl.BlockSpec((BN, DP), lambda i: (i, 0)),
            pl.BlockSpec((1, DP), lambda i: (0, 0)),
        ],
        out_specs=pl.BlockSpec((BN, DP), lambda i: (i, 0)),
        out_shape=jax.ShapeDtypeStruct((NP, DP), jnp.float32),
    )(agg, selfloop, bias_p)


def kernel(entity, edge_index, edge_type, emb_table,
           comp1, basis1, root1, bias1, comp2, basis2, root2, bias2):
    entity = entity.astype(jnp.int32)
    edge_index = edge_index.astype(jnp.int32)
    edge_type = edge_type.astype(jnp.int32)

    pad_e = EP - E
    src_p = jnp.concatenate([edge_index[0], jnp.zeros((pad_e,), jnp.int32)])
    dst_p = jnp.concatenate([edge_index[1],
                             jnp.full((pad_e,), DUMP, jnp.int32)])
    t_p = jnp.concatenate([edge_type, jnp.zeros((pad_e,), jnp.int32)])
    ent_p = jnp.concatenate([entity, jnp.zeros((NP - N,), jnp.int32)])
    emb_p = jnp.pad(emb_table, ((0, 0), (0, DP - D)))
    zeros_blk = jnp.zeros((AGG_W, DP), jnp.float32)
    bias1_p = jnp.concatenate([bias1, jnp.zeros((DP - D,), jnp.float32)])
    bias1_p = bias1_p.reshape(1, DP)
    bias2_p = jnp.concatenate([bias2, jnp.zeros((DP - D,), jnp.float32)])
    bias2_p = bias2_p.reshape(1, DP)

    x0, invcnt = _sc_lookup_counts(ent_p, emb_p, dst_p, t_p)

    w1 = _wbuild(comp1, basis1, root1)
    w1 = jnp.concatenate([w1, jnp.zeros((TR, DP - D, DP), jnp.float32)],
                         axis=1)
    xw1 = _xw_matmul(x0, w1)
    xw1_rows = xw1.reshape(TR * NP, DP)
    agg1 = _sc_edge_agg(xw1_rows, invcnt, zeros_blk, src_p, dst_p, t_p)

    w2 = _wbuild(comp2, basis2, root2)
    w2 = jnp.concatenate([w2, jnp.zeros((TR, DP - D, DP), jnp.float32)],
                         axis=1)
    xw2 = _combine_matmul(agg1, xw1[R], bias1_p, w2)
    xw2_rows = xw2.reshape(TR * NP, DP)
    agg2 = _sc_edge_agg(xw2_rows, invcnt, zeros_blk, src_p, dst_p, t_p)

    out = _combine_final(agg2, xw2[R], bias2_p)
    return out[:N, :D]


# speedup vs baseline: 1.9651x; 1.0004x over previous
import functools

import jax
import jax.numpy as jnp
from jax import lax
from jax.experimental import pallas as pl
from jax.experimental.pallas import tpu as pltpu
from jax.experimental.pallas import tpu_sc as plsc

N = 10000
E = 160000
D = 100
R = 20
NB = 4

NC = 2
NS = 16
NW = NC * NS

NP = 10240
ROWS_W = NP // NW
LCH = 64
EP = 163840
EW = EP // NW
EK = 128
NCHUNK = EW // EK
NRP = 200704
SEGW = NRP // NW
DP = 128
TR = R + 1
DUMP = N
AGG_W = NP // NS

_mesh = plsc.VectorSubcoreMesh(core_axis_name="c", subcore_axis_name="s")


def _sc_lookup_counts_body(entity_ref, emb_ref, dst_ref, t_ref,
                           x_out, inv_out,
                           idx_v, rows_v, ebuf_d, ebuf_t, cnt_v, sem):
    c = lax.axis_index("c")
    s = lax.axis_index("s")
    w = s * NC + c

    pltpu.sync_copy(entity_ref.at[pl.ds(w * ROWS_W, ROWS_W)], idx_v)
    for k in range(ROWS_W // LCH):
        pltpu.async_copy(emb_ref.at[idx_v.at[pl.ds(k * LCH, LCH)]],
                         rows_v.at[pl.ds(k * LCH, LCH)], sem).wait()
    pltpu.sync_copy(rows_v, x_out.at[pl.ds(w * ROWS_W, ROWS_W)])

    def zero_body(i, _):
        cnt_v[pl.ds(i * 16, 16)] = jnp.zeros((16,), jnp.float32)
        return 0
    lax.fori_loop(0, SEGW // 16, zero_body, 0, unroll=4)

    lo = w * SEGW
    ones16 = jnp.ones((16,), jnp.float32)

    def chunk_body(ci, _):
        pltpu.sync_copy(dst_ref.at[pl.ds(ci * EW, EW)], ebuf_d)
        pltpu.sync_copy(t_ref.at[pl.ds(ci * EW, EW)], ebuf_t)

        def vec_body(vi, _):
            d16 = ebuf_d[pl.ds(vi * 16, 16)]
            t16 = ebuf_t[pl.ds(vi * 16, 16)]
            rel = d16 * R + t16 - lo
            m = (rel >= 0) & (rel < SEGW)
            relc = jnp.where(m, rel, 0)
            plsc.addupdate_scatter(cnt_v, [relc], ones16, mask=m)
            return 0
        lax.fori_loop(0, EW // 16, vec_body, 0, unroll=4)
        return 0
    lax.fori_loop(0, EP // EW, chunk_body, 0)

    def inv_body(i, _):
        v = cnt_v[pl.ds(i * 16, 16)]
        cnt_v[pl.ds(i * 16, 16)] = 1.0 / jnp.maximum(v, 1.0)
        return 0
    lax.fori_loop(0, SEGW // 16, inv_body, 0, unroll=4)
    pltpu.sync_copy(cnt_v, inv_out.at[pl.ds(lo, SEGW)])


_sc_lookup_counts = pl.kernel(
    _sc_lookup_counts_body,
    out_type=(jax.ShapeDtypeStruct((NP, DP), jnp.float32),
              jax.ShapeDtypeStruct((NRP,), jnp.float32)),
    mesh=_mesh,
    scratch_types=[
        pltpu.VMEM((ROWS_W,), jnp.int32),
        pltpu.VMEM((ROWS_W, DP), jnp.float32),
        pltpu.VMEM((EW,), jnp.int32),
        pltpu.VMEM((EW,), jnp.int32),
        pltpu.VMEM((SEGW,), jnp.float32),
        pltpu.SemaphoreType.DMA,
    ],
    compiler_params=pltpu.CompilerParams(needs_layout_passes=False),
)


def _sc_edge_agg_body(xw_ref, inv_ref, zeros_ref, src_ref, dst_ref, t_ref,
                      out_ref,
                      ebuf_s, ebuf_d, ebuf_t, gidx, sidx, didx,
                      rows_v, s_v, agg_sh, sem, sem2):
    c = lax.axis_index("c")
    s = lax.axis_index("s")
    w = s * NC + c

    pltpu.sync_copy(zeros_ref, agg_sh.at[pl.ds(s * AGG_W, AGG_W)])
    plsc.subcore_barrier()

    pltpu.sync_copy(src_ref.at[pl.ds(w * EW, EW)], ebuf_s)
    pltpu.sync_copy(dst_ref.at[pl.ds(w * EW, EW)], ebuf_d)
    pltpu.sync_copy(t_ref.at[pl.ds(w * EW, EW)], ebuf_t)

    def chunk_body(ci, _):
        def bidx(vi, _):
            s16 = ebuf_s[pl.ds(ci * EK + vi * 16, 16)]
            d16 = ebuf_d[pl.ds(ci * EK + vi * 16, 16)]
            t16 = ebuf_t[pl.ds(ci * EK + vi * 16, 16)]
            gidx[pl.ds(vi * 16, 16)] = t16 * NP + s16
            sidx[pl.ds(vi * 16, 16)] = d16 * R + t16
            didx[pl.ds(vi * 16, 16)] = d16
            return 0
        lax.fori_loop(0, EK // 16, bidx, 0, unroll=8)

        cp_rows = pltpu.async_copy(xw_ref.at[gidx], rows_v, sem)
        cp_s = pltpu.async_copy(inv_ref.at[sidx], s_v, sem2)
        cp_rows.wait()
        cp_s.wait()

        def scale_row(r, _):
            sc = plsc.load_gather(s_v, [jnp.full((16,), r, jnp.int32)])
            for g in range(DP // 16):
                v = rows_v[r, pl.ds(g * 16, 16)]
                rows_v[r, pl.ds(g * 16, 16)] = v * sc
            return 0
        lax.fori_loop(0, EK, scale_row, 0)

        pltpu.sync_copy(rows_v, agg_sh.at[didx], add=True)
        return 0
    lax.fori_loop(0, NCHUNK, chunk_body, 0)

    plsc.subcore_barrier()
    pltpu.sync_copy(agg_sh.at[pl.ds(s * AGG_W, AGG_W)],
                    out_ref.at[c, pl.ds(s * AGG_W, AGG_W)])


_sc_edge_agg = pl.kernel(
    _sc_edge_agg_body,
    out_type=jax.ShapeDtypeStruct((NC, NP, DP), jnp.float32),
    mesh=_mesh,
    scratch_types=[
        pltpu.VMEM((EW,), jnp.int32),
        pltpu.VMEM((EW,), jnp.int32),
        pltpu.VMEM((EW,), jnp.int32),
        pltpu.VMEM((EK,), jnp.int32),
        pltpu.VMEM((EK,), jnp.int32),
        pltpu.VMEM((EK,), jnp.int32),
        pltpu.VMEM((EK, DP), jnp.float32),
        pltpu.VMEM((EK,), jnp.float32),
        pltpu.MemorySpace.VMEM_SHARED((NP, DP), jnp.float32),
        pltpu.SemaphoreType.DMA,
        pltpu.SemaphoreType.DMA,
    ],
    compiler_params=pltpu.CompilerParams(needs_layout_passes=False),
)


def _wbuild_body(comp_ref, basis_ref, root_ref, w_out):
    basis2d = jnp.concatenate(
        [basis_ref[...], jnp.zeros((NB, D, DP - D), jnp.float32)], axis=2
    ).reshape(NB, D * DP)
    w = lax.dot_general(comp_ref[...], basis2d, (((1,), (0,)), ((), ())),
                        preferred_element_type=jnp.float32)
    rootp = jnp.concatenate(
        [root_ref[...], jnp.zeros((D, DP - D), jnp.float32)], axis=1
    ).reshape(1, D * DP)
    w_out[...] = jnp.concatenate([w, rootp], axis=0).reshape(TR, D, DP)


def _wbuild(comp, basis, root):
    return pl.pallas_call(
        _wbuild_body,
        out_shape=jax.ShapeDtypeStruct((TR, D, DP), jnp.float32),
    )(comp, basis, root)


BN = 1024


def _xw_body(x_ref, w_ref, out_ref):
    out_ref[...] = lax.dot_general(
        x_ref[...], w_ref[0], (((1,), (0,)), ((), ())),
        preferred_element_type=jnp.float32)[None, :, :]


def _xw_matmul(x, w):
    k = x.shape[1]
    return pl.pallas_call(
        _xw_body,
        grid=(NP // BN, TR),
        in_specs=[
            pl.BlockSpec((BN, k), lambda i, j: (i, 0)),
            pl.BlockSpec((1, k, DP), lambda i, j: (j, 0, 0)),
        ],
        out_specs=pl.BlockSpec((1, BN, DP), lambda i, j: (j, i, 0)),
        out_shape=jax.ShapeDtypeStruct((TR, NP, DP), jnp.float32),
    )(x, w)


def _combine_matmul_body(agg_ref, self_ref, bias_ref, w_ref, out_ref):
    h = agg_ref[0] + agg_ref[1] + self_ref[...] + bias_ref[...]
    h = jnp.maximum(h, 0.0)
    out_ref[...] = lax.dot_general(
        h, w_ref[0], (((1,), (0,)), ((), ())),
        preferred_element_type=jnp.float32)[None, :, :]


def _combine_matmul(agg, selfloop, bias_p, w):
    return pl.pallas_call(
        _combine_matmul_body,
        grid=(NP // BN, TR),
        in_specs=[
            pl.BlockSpec((NC, BN, DP), lambda i, j: (0, i, 0)),
            pl.BlockSpec((BN, DP), lambda i, j: (i, 0)),
            pl.BlockSpec((1, DP), lambda i, j: (0, 0)),
            pl.BlockSpec((1, DP, DP), lambda i, j: (j, 0, 0)),
        ],
        out_specs=pl.BlockSpec((1, BN, DP), lambda i, j: (j, i, 0)),
        out_shape=jax.ShapeDtypeStruct((TR, NP, DP), jnp.float32),
    )(agg, selfloop, bias_p, w)


def _combine_final_body(agg_ref, self_ref, bias_ref, out_ref):
    out_ref[...] = agg_ref[0] + agg_ref[1] + self_ref[...] + bias_ref[...]


def _combine_final(agg, selfloop, bias_p):
    return pl.pallas_call(
        _combine_final_body,
        grid=(NP // BN,),
        in_specs=[
            pl.BlockSpec((NC, BN, DP), lambda i: (0, i, 0)),
            pl.BlockSpec((BN, DP), lambda i: (i, 0)),
            pl.BlockSpec((1, DP), lambda i: (0, 0)),
        ],
        out_specs=pl.BlockSpec((BN, DP), lambda i: (i, 0)),
        out_shape=jax.ShapeDtypeStruct((NP, DP), jnp.float32),
    )(agg, selfloop, bias_p)


def kernel(entity, edge_index, edge_type, emb_table,
           comp1, basis1, root1, bias1, comp2, basis2, root2, bias2):
    entity = entity.astype(jnp.int32)
    edge_index = edge_index.astype(jnp.int32)
    edge_type = edge_type.astype(jnp.int32)

    pad_e = EP - E
    src_p = jnp.concatenate([edge_index[0], jnp.zeros((pad_e,), jnp.int32)])
    dst_p = jnp.concatenate([edge_index[1],
                             jnp.full((pad_e,), DUMP, jnp.int32)])
    t_p = jnp.concatenate([edge_type, jnp.zeros((pad_e,), jnp.int32)])
    ent_p = jnp.concatenate([entity, jnp.zeros((NP - N,), jnp.int32)])
    emb_p = jnp.pad(emb_table, ((0, 0), (0, DP - D)))
    zeros_blk = jnp.zeros((AGG_W, DP), jnp.float32)
    bias1_p = jnp.concatenate([bias1, jnp.zeros((DP - D,), jnp.float32)])
    bias1_p = bias1_p.reshape(1, DP)
    bias2_p = jnp.concatenate([bias2, jnp.zeros((DP - D,), jnp.float32)])
    bias2_p = bias2_p.reshape(1, DP)

    x0, invcnt = _sc_lookup_counts(ent_p, emb_p, dst_p, t_p)

    w1 = _wbuild(comp1, basis1, root1)
    w1 = jnp.concatenate([w1, jnp.zeros((TR, DP - D, DP), jnp.float32)],
                         axis=1)
    xw1 = _xw_matmul(x0, w1)
    xw1_rows = xw1.reshape(TR * NP, DP)
    agg1 = _sc_edge_agg(xw1_rows, invcnt, zeros_blk, src_p, dst_p, t_p)

    w2 = _wbuild(comp2, basis2, root2)
    w2 = jnp.concatenate([w2, jnp.zeros((TR, DP - D, DP), jnp.float32)],
                         axis=1)
    xw2 = _combine_matmul(agg1, xw1[R], bias1_p, w2)
    xw2_rows = xw2.reshape(TR * NP, DP)
    agg2 = _sc_edge_agg(xw2_rows, invcnt, zeros_blk, src_p, dst_p, t_p)

    out = _combine_final(agg2, xw2[R], bias2_p)
    return out[:N, :D]


# speedup vs baseline: 2.3450x; 1.1933x over previous
import jax
import jax.numpy as jnp
from jax import lax
from jax.experimental import pallas as pl
from jax.experimental.pallas import tpu as pltpu
from jax.experimental.pallas import tpu_sc as plsc

N = 10000
E = 160000
D = 100
R = 20
NB = 4
NE = 100000

NC = 2
NS = 16
NW = NC * NS

NP = 10240
ROWS_W = NP // NW
LCH = 64
EP = 163840
EW = EP // NW
EK = 128
NCHUNK = EW // EK
NRP = 200704
SEGW = NRP // NW
DP = 128
TR = R + 1
DUMP = N
AGG_W = NP // NS

_mesh = plsc.VectorSubcoreMesh(core_axis_name="c", subcore_axis_name="s")
_sc_params = pltpu.CompilerParams(needs_layout_passes=False)


def _sc_lookup_body(entity_ref, emb_ref, x_out, idx_v, rows_v, sem):
    c = lax.axis_index("c")
    s = lax.axis_index("s")
    w = s * NC + c
    pltpu.sync_copy(entity_ref.at[pl.ds(w * ROWS_W, ROWS_W)], idx_v)
    cps = [
        pltpu.async_copy(emb_ref.at[idx_v.at[pl.ds(k * LCH, LCH)]],
                         rows_v.at[pl.ds(k * LCH, LCH)], sem)
        for k in range(ROWS_W // LCH)
    ]
    for cp in cps:
        cp.wait()
    pltpu.sync_copy(rows_v, x_out.at[pl.ds(w * ROWS_W, ROWS_W)])


_sc_lookup = pl.kernel(
    _sc_lookup_body,
    out_type=jax.ShapeDtypeStruct((NP, DP), jnp.float32),
    mesh=_mesh,
    scratch_types=[
        pltpu.VMEM((ROWS_W,), jnp.int32),
        pltpu.VMEM((ROWS_W, DP), jnp.float32),
        pltpu.SemaphoreType.DMA,
    ],
    compiler_params=_sc_params,
)


def _sc_counts_body(dst_ref, t_ref, inv_out, ebuf_d, ebuf_t, cnt_v):
    c = lax.axis_index("c")
    s = lax.axis_index("s")
    w = s * NC + c

    def zero_body(i, _):
        cnt_v[pl.ds(i * 16, 16)] = jnp.zeros((16,), jnp.float32)
        return 0
    lax.fori_loop(0, SEGW // 16, zero_body, 0, unroll=4)

    lo = w * SEGW
    ones16 = jnp.ones((16,), jnp.float32)

    def chunk_body(ci, _):
        pltpu.sync_copy(dst_ref.at[pl.ds(ci * EW, EW)], ebuf_d)
        pltpu.sync_copy(t_ref.at[pl.ds(ci * EW, EW)], ebuf_t)

        def vec_body(vi, _):
            d16 = ebuf_d[pl.ds(vi * 16, 16)]
            t16 = ebuf_t[pl.ds(vi * 16, 16)]
            rel = d16 * R + t16 - lo
            m = (rel >= 0) & (rel < SEGW)
            relc = jnp.where(m, rel, 0)
            plsc.addupdate_scatter(cnt_v, [relc], ones16, mask=m)
            return 0
        lax.fori_loop(0, EW // 16, vec_body, 0, unroll=4)
        return 0
    lax.fori_loop(0, EP // EW, chunk_body, 0)

    def inv_body(i, _):
        v = cnt_v[pl.ds(i * 16, 16)]
        cnt_v[pl.ds(i * 16, 16)] = 1.0 / jnp.maximum(v, 1.0)
        return 0
    lax.fori_loop(0, SEGW // 16, inv_body, 0, unroll=4)
    pltpu.sync_copy(cnt_v, inv_out.at[pl.ds(lo, SEGW)])


_sc_counts = pl.kernel(
    _sc_counts_body,
    out_type=jax.ShapeDtypeStruct((NRP,), jnp.float32),
    mesh=_mesh,
    scratch_types=[
        pltpu.VMEM((EW,), jnp.int32),
        pltpu.VMEM((EW,), jnp.int32),
        pltpu.VMEM((SEGW,), jnp.float32),
    ],
    compiler_params=_sc_params,
)


def _sc_edge_agg_body(xw_ref, inv_ref, zeros_ref, src_ref, dst_ref, t_ref,
                      out_ref,
                      ebuf_s, ebuf_d, ebuf_t, gidx, sidx, didx,
                      rows_v, s_v, agg_sh, sem, sem2):
    c = lax.axis_index("c")
    s = lax.axis_index("s")
    w = s * NC + c

    pltpu.sync_copy(zeros_ref, agg_sh.at[pl.ds(s * AGG_W, AGG_W)])
    plsc.subcore_barrier()

    pltpu.sync_copy(src_ref.at[pl.ds(w * EW, EW)], ebuf_s)
    pltpu.sync_copy(dst_ref.at[pl.ds(w * EW, EW)], ebuf_d)
    pltpu.sync_copy(t_ref.at[pl.ds(w * EW, EW)], ebuf_t)

    def chunk_body(ci, _):
        def bidx(vi, _):
            s16 = ebuf_s[pl.ds(ci * EK + vi * 16, 16)]
            d16 = ebuf_d[pl.ds(ci * EK + vi * 16, 16)]
            t16 = ebuf_t[pl.ds(ci * EK + vi * 16, 16)]
            gidx[pl.ds(vi * 16, 16)] = s16 * TR + t16
            sidx[pl.ds(vi * 16, 16)] = d16 * R + t16
            didx[pl.ds(vi * 16, 16)] = d16
            return 0
        lax.fori_loop(0, EK // 16, bidx, 0, unroll=8)

        cp_rows = pltpu.async_copy(xw_ref.at[gidx], rows_v, sem)
        cp_s = pltpu.async_copy(inv_ref.at[sidx], s_v, sem2)
        cp_rows.wait()
        cp_s.wait()

        def scale_row(r, _):
            sc = plsc.load_gather(s_v, [jnp.full((16,), r, jnp.int32)])
            for g in range(DP // 16):
                v = rows_v[r, pl.ds(g * 16, 16)]
                rows_v[r, pl.ds(g * 16, 16)] = v * sc
            return 0
        lax.fori_loop(0, EK, scale_row, 0)

        pltpu.sync_copy(rows_v, agg_sh.at[didx], add=True)
        return 0
    lax.fori_loop(0, NCHUNK, chunk_body, 0)

    plsc.subcore_barrier()
    pltpu.sync_copy(agg_sh.at[pl.ds(s * AGG_W, AGG_W)],
                    out_ref.at[c, pl.ds(s * AGG_W, AGG_W)])


_sc_edge_agg = pl.kernel(
    _sc_edge_agg_body,
    out_type=jax.ShapeDtypeStruct((NC, NP, DP), jnp.float32),
    mesh=_mesh,
    scratch_types=[
        pltpu.VMEM((EW,), jnp.int32),
        pltpu.VMEM((EW,), jnp.int32),
        pltpu.VMEM((EW,), jnp.int32),
        pltpu.VMEM((EK,), jnp.int32),
        pltpu.VMEM((EK,), jnp.int32),
        pltpu.VMEM((EK,), jnp.int32),
        pltpu.VMEM((EK, DP), jnp.float32),
        pltpu.VMEM((EK,), jnp.float32),
        pltpu.MemorySpace.VMEM_SHARED((NP, DP), jnp.float32),
        pltpu.SemaphoreType.DMA,
        pltpu.SemaphoreType.DMA,
    ],
    compiler_params=_sc_params,
)


EB = 2000


def _pad_emb_body(emb_ref, out_ref):
    out_ref[...] = jnp.concatenate(
        [emb_ref[...], jnp.zeros((EB, DP - D), jnp.float32)], axis=1)


def _pad_emb(emb):
    return pl.pallas_call(
        _pad_emb_body,
        grid=(NE // EB,),
        in_specs=[pl.BlockSpec((EB, D), lambda i: (i, 0))],
        out_specs=pl.BlockSpec((EB, DP), lambda i: (i, 0)),
        out_shape=jax.ShapeDtypeStruct((NE, DP), jnp.float32),
    )(emb)


def _wbuild_body(comp_ref, basis_ref, root_ref, w_out):
    basis2d = jnp.pad(basis_ref[...],
                      ((0, 0), (0, DP - D), (0, DP - D))).reshape(NB, DP * DP)
    w = lax.dot_general(comp_ref[...], basis2d, (((1,), (0,)), ((), ())),
                        preferred_element_type=jnp.float32)
    rootp = jnp.pad(root_ref[...], ((0, DP - D), (0, DP - D))).reshape(1, DP * DP)
    w = jnp.concatenate([w, rootp], axis=0).reshape(TR, DP, DP)
    w_out[...] = w.transpose(1, 0, 2).reshape(DP, TR * DP)


def _wbuild(comp, basis, root):
    return pl.pallas_call(
        _wbuild_body,
        out_shape=jax.ShapeDtypeStruct((DP, TR * DP), jnp.float32),
    )(comp, basis, root)


BN = 1024


def _xw_body(x_ref, w_ref, out_ref):
    out_ref[...] = lax.dot_general(
        x_ref[...], w_ref[...], (((1,), (0,)), ((), ())),
        preferred_element_type=jnp.float32)


def _xw_matmul(x, w):
    return pl.pallas_call(
        _xw_body,
        grid=(NP // BN,),
        in_specs=[
            pl.BlockSpec((BN, DP), lambda i: (i, 0)),
            pl.BlockSpec((DP, TR * DP), lambda i: (0, 0)),
        ],
        out_specs=pl.BlockSpec((BN, TR * DP), lambda i: (i, 0)),
        out_shape=jax.ShapeDtypeStruct((NP, TR * DP), jnp.float32),
    )(x, w)


def _combine_matmul_body(agg_ref, self_ref, bias_ref, w_ref, out_ref):
    h = agg_ref[0] + agg_ref[1] + self_ref[...] + bias_ref[...]
    h = jnp.maximum(h, 0.0)
    out_ref[...] = lax.dot_general(
        h, w_ref[...], (((1,), (0,)), ((), ())),
        preferred_element_type=jnp.float32)


def _combine_matmul(agg, selfloop, bias_p, w):
    return pl.pallas_call(
        _combine_matmul_body,
        grid=(NP // BN,),
        in_specs=[
            pl.BlockSpec((NC, BN, DP), lambda i: (0, i, 0)),
            pl.BlockSpec((BN, DP), lambda i: (i, 0)),
            pl.BlockSpec((1, DP), lambda i: (0, 0)),
            pl.BlockSpec((DP, TR * DP), lambda i: (0, 0)),
        ],
        out_specs=pl.BlockSpec((BN, TR * DP), lambda i: (i, 0)),
        out_shape=jax.ShapeDtypeStruct((NP, TR * DP), jnp.float32),
    )(agg, selfloop, bias_p, w)


def _combine_final_body(agg_ref, self_ref, bias_ref, out_ref):
    out_ref[...] = agg_ref[0] + agg_ref[1] + self_ref[...] + bias_ref[...]


def _combine_final(agg, selfloop, bias_p):
    return pl.pallas_call(
        _combine_final_body,
        grid=(NP // BN,),
        in_specs=[
            pl.BlockSpec((NC, BN, DP), lambda i: (0, i, 0)),
            pl.BlockSpec((BN, DP), lambda i: (i, 0)),
            pl.BlockSpec((1, DP), lambda i: (0, 0)),
        ],
        out_specs=pl.BlockSpec((BN, DP), lambda i: (i, 0)),
        out_shape=jax.ShapeDtypeStruct((NP, DP), jnp.float32),
    )(agg, selfloop, bias_p)


def kernel(entity, edge_index, edge_type, emb_table,
           comp1, basis1, root1, bias1, comp2, basis2, root2, bias2):
    entity = entity.astype(jnp.int32)
    edge_index = edge_index.astype(jnp.int32)
    edge_type = edge_type.astype(jnp.int32)

    pad_e = EP - E
    src_p = jnp.concatenate([edge_index[0], jnp.zeros((pad_e,), jnp.int32)])
    dst_p = jnp.concatenate([edge_index[1],
                             jnp.full((pad_e,), DUMP, jnp.int32)])
    t_p = jnp.concatenate([edge_type, jnp.zeros((pad_e,), jnp.int32)])
    ent_p = jnp.concatenate([entity, jnp.zeros((NP - N,), jnp.int32)])
    zeros_blk = jnp.zeros((AGG_W, DP), jnp.float32)
    bias1_p = jnp.concatenate([bias1, jnp.zeros((DP - D,), jnp.float32)])
    bias1_p = bias1_p.reshape(1, DP)
    bias2_p = jnp.concatenate([bias2, jnp.zeros((DP - D,), jnp.float32)])
    bias2_p = bias2_p.reshape(1, DP)

    emb_p = _pad_emb(emb_table)
    x0 = _sc_lookup(ent_p, emb_p)
    invcnt = _sc_counts(dst_p, t_p)

    w1 = _wbuild(comp1, basis1, root1)
    xw1 = _xw_matmul(x0, w1)
    xw1_rows = xw1.reshape(NP * TR, DP)
    agg1 = _sc_edge_agg(xw1_rows, invcnt, zeros_blk, src_p, dst_p, t_p)

    w2 = _wbuild(comp2, basis2, root2)
    xw2 = _combine_matmul(agg1, xw1[:, R * DP:], bias1_p, w2)
    xw2_rows = xw2.reshape(NP * TR, DP)
    agg2 = _sc_edge_agg(xw2_rows, invcnt, zeros_blk, src_p, dst_p, t_p)

    out = _combine_final(agg2, xw2[:, R * DP:], bias2_p)
    return out[:N, :D]


# speedup vs baseline: 3.6169x; 1.5424x over previous
import jax
import jax.numpy as jnp
from jax import lax
from jax.experimental import pallas as pl
from jax.experimental.pallas import tpu as pltpu
from jax.experimental.pallas import tpu_sc as plsc

N = 10000
E = 160000
D = 100
R = 20
NB = 4
NE = 100000

NC = 2
NS = 16
NW = NC * NS

NP = 10240
ROWS_W = NP // NW
LCH = 64
EP = 163840
EW = EP // NW
EK = 128
NCHUNK = EW // EK
NPAIR = NCHUNK // 2
NRP = 200704
SEGW = NRP // NW
CNT_W = NRP // NS
DP = 128
TR = R + 1
DUMP = N
AGG_W = NP // NS

_mesh = plsc.VectorSubcoreMesh(core_axis_name="c", subcore_axis_name="s")
_sc_params = pltpu.CompilerParams(needs_layout_passes=False)


def _sc_lookup_body(entity_ref, emb_ref, x_out, idx_v, rows_v, sem):
    c = lax.axis_index("c")
    s = lax.axis_index("s")
    w = s * NC + c
    pltpu.sync_copy(entity_ref.at[pl.ds(w * ROWS_W, ROWS_W)], idx_v)
    cps = [
        pltpu.async_copy(emb_ref.at[idx_v.at[pl.ds(k * LCH, LCH)]],
                         rows_v.at[pl.ds(k * LCH, LCH)], sem)
        for k in range(ROWS_W // LCH)
    ]
    for cp in cps:
        cp.wait()
    pltpu.sync_copy(rows_v, x_out.at[pl.ds(w * ROWS_W, ROWS_W)])


_sc_lookup = pl.kernel(
    _sc_lookup_body,
    out_type=jax.ShapeDtypeStruct((NP, DP), jnp.float32),
    mesh=_mesh,
    scratch_types=[
        pltpu.VMEM((ROWS_W,), jnp.int32),
        pltpu.VMEM((ROWS_W, DP), jnp.float32),
        pltpu.SemaphoreType.DMA,
    ],
    compiler_params=_sc_params,
)


def _sc_counts_body(dst_ref, t_ref, zeros_ref, part_out,
                    ebuf_d, ebuf_t, segbuf, ones_v, cnt_sh):
    c = lax.axis_index("c")
    s = lax.axis_index("s")
    w = s * NC + c

    pltpu.sync_copy(zeros_ref, cnt_sh.at[pl.ds(s * CNT_W, CNT_W)])

    def ones_body(i, _):
        ones_v[pl.ds(i * 16, 16)] = jnp.ones((16,), jnp.float32)
        return 0
    lax.fori_loop(0, EK // 16, ones_body, 0, unroll=8)

    pltpu.sync_copy(dst_ref.at[pl.ds(w * EW, EW)], ebuf_d)
    pltpu.sync_copy(t_ref.at[pl.ds(w * EW, EW)], ebuf_t)
    plsc.subcore_barrier()

    def chunk_body(ci, _):
        def bidx(vi, _):
            d16 = ebuf_d[pl.ds(ci * EK + vi * 16, 16)]
            t16 = ebuf_t[pl.ds(ci * EK + vi * 16, 16)]
            segbuf[pl.ds(vi * 16, 16)] = d16 * R + t16
            return 0
        lax.fori_loop(0, EK // 16, bidx, 0, unroll=8)
        pltpu.sync_copy(ones_v, cnt_sh.at[segbuf], add=True)
        return 0
    lax.fori_loop(0, NCHUNK, chunk_body, 0)

    plsc.subcore_barrier()
    pltpu.sync_copy(cnt_sh.at[pl.ds(s * CNT_W, CNT_W)],
                    part_out.at[c, pl.ds(s * CNT_W, CNT_W)])


_sc_counts = pl.kernel(
    _sc_counts_body,
    out_type=jax.ShapeDtypeStruct((NC, NRP), jnp.float32),
    mesh=_mesh,
    scratch_types=[
        pltpu.VMEM((EW,), jnp.int32),
        pltpu.VMEM((EW,), jnp.int32),
        pltpu.VMEM((EK,), jnp.int32),
        pltpu.VMEM((EK,), jnp.float32),
        pltpu.MemorySpace.VMEM_SHARED((NRP,), jnp.float32),
    ],
    compiler_params=_sc_params,
)


def _sc_inv_body(part_ref, inv_out, buf_a, buf_b):
    c = lax.axis_index("c")
    s = lax.axis_index("s")
    w = s * NC + c
    lo = w * SEGW
    pltpu.sync_copy(part_ref.at[0, pl.ds(lo, SEGW)], buf_a)
    pltpu.sync_copy(part_ref.at[1, pl.ds(lo, SEGW)], buf_b)

    def inv_body(i, _):
        v = buf_a[pl.ds(i * 16, 16)] + buf_b[pl.ds(i * 16, 16)]
        buf_a[pl.ds(i * 16, 16)] = 1.0 / jnp.maximum(v, 1.0)
        return 0
    lax.fori_loop(0, SEGW // 16, inv_body, 0, unroll=8)
    pltpu.sync_copy(buf_a, inv_out.at[pl.ds(lo, SEGW)])


_sc_inv = pl.kernel(
    _sc_inv_body,
    out_type=jax.ShapeDtypeStruct((NRP,), jnp.float32),
    mesh=_mesh,
    scratch_types=[
        pltpu.VMEM((SEGW,), jnp.float32),
        pltpu.VMEM((SEGW,), jnp.float32),
    ],
    compiler_params=_sc_params,
)


def _sc_edge_agg_body(xw_ref, inv_ref, zeros_ref, src_ref, dst_ref, t_ref,
                      out_ref,
                      ebuf_s, ebuf_d, ebuf_t,
                      gidx_a, sidx_a, didx_a, rows_a, sv_a,
                      gidx_b, sidx_b, didx_b, rows_b, sv_b,
                      agg_sh, sem_a, sem_b, sem_sa, sem_sb):
    c = lax.axis_index("c")
    s = lax.axis_index("s")
    w = s * NC + c

    pltpu.sync_copy(zeros_ref, agg_sh.at[pl.ds(s * AGG_W, AGG_W)])
    plsc.subcore_barrier()

    pltpu.sync_copy(src_ref.at[pl.ds(w * EW, EW)], ebuf_s)
    pltpu.sync_copy(dst_ref.at[pl.ds(w * EW, EW)], ebuf_d)
    pltpu.sync_copy(t_ref.at[pl.ds(w * EW, EW)], ebuf_t)

    def build(ci, gidx, sidx, didx):
        def bidx(vi, _):
            s16 = ebuf_s[pl.ds(ci * EK + vi * 16, 16)]
            d16 = ebuf_d[pl.ds(ci * EK + vi * 16, 16)]
            t16 = ebuf_t[pl.ds(ci * EK + vi * 16, 16)]
            gidx[pl.ds(vi * 16, 16)] = t16 * NP + s16
            sidx[pl.ds(vi * 16, 16)] = d16 * R + t16
            didx[pl.ds(vi * 16, 16)] = d16
            return 0
        lax.fori_loop(0, EK // 16, bidx, 0, unroll=8)

    def fire(gidx, sidx, rows, sv, sem):
        pltpu.async_copy(xw_ref.at[gidx], rows, sem)
        pltpu.async_copy(inv_ref.at[sidx], sv, sem)

    def wait_gather(gidx, sidx, rows, sv, sem):
        pltpu.make_async_copy(xw_ref.at[gidx], rows, sem).wait()
        pltpu.make_async_copy(inv_ref.at[sidx], sv, sem).wait()

    def scale(rows, sv):
        def scale_row(r, _):
            sc = plsc.load_gather(sv, [jnp.full((16,), r, jnp.int32)])
            for g in range(DP // 16):
                v = rows[r, pl.ds(g * 16, 16)]
                rows[r, pl.ds(g * 16, 16)] = v * sc
            return 0
        lax.fori_loop(0, EK, scale_row, 0, unroll=2)

    def fire_scatter(rows, didx, sem):
        pltpu.async_copy(rows, agg_sh.at[didx], sem, add=True)

    def wait_scatter(rows, didx, sem):
        pltpu.make_async_copy(rows, agg_sh.at[didx], sem).wait()

    build(0, gidx_a, sidx_a, didx_a)
    fire(gidx_a, sidx_a, rows_a, sv_a, sem_a)

    def pair_body(ci, _):

        @pl.when(ci > 0)
        def _():
            wait_scatter(rows_b, didx_b, sem_sb)

        build(2 * ci + 1, gidx_b, sidx_b, didx_b)
        fire(gidx_b, sidx_b, rows_b, sv_b, sem_b)

        wait_gather(gidx_a, sidx_a, rows_a, sv_a, sem_a)
        scale(rows_a, sv_a)
        fire_scatter(rows_a, didx_a, sem_sa)
        wait_scatter(rows_a, didx_a, sem_sa)

        @pl.when(ci < NPAIR - 1)
        def _():
            build(2 * ci + 2, gidx_a, sidx_a, didx_a)
            fire(gidx_a, sidx_a, rows_a, sv_a, sem_a)

        wait_gather(gidx_b, sidx_b, rows_b, sv_b, sem_b)
        scale(rows_b, sv_b)
        fire_scatter(rows_b, didx_b, sem_sb)
        return 0
    lax.fori_loop(0, NPAIR, pair_body, 0)
    wait_scatter(rows_b, didx_b, sem_sb)

    plsc.subcore_barrier()
    pltpu.sync_copy(agg_sh.at[pl.ds(s * AGG_W, AGG_W)],
                    out_ref.at[c, pl.ds(s * AGG_W, AGG_W)])


_sc_edge_agg = pl.kernel(
    _sc_edge_agg_body,
    out_type=jax.ShapeDtypeStruct((NC, NP, DP), jnp.float32),
    mesh=_mesh,
    scratch_types=[
        pltpu.VMEM((EW,), jnp.int32),
        pltpu.VMEM((EW,), jnp.int32),
        pltpu.VMEM((EW,), jnp.int32),
        pltpu.VMEM((EK,), jnp.int32),
        pltpu.VMEM((EK,), jnp.int32),
        pltpu.VMEM((EK,), jnp.int32),
        pltpu.VMEM((EK, DP), jnp.float32),
        pltpu.VMEM((EK,), jnp.float32),
        pltpu.VMEM((EK,), jnp.int32),
        pltpu.VMEM((EK,), jnp.int32),
        pltpu.VMEM((EK,), jnp.int32),
        pltpu.VMEM((EK, DP), jnp.float32),
        pltpu.VMEM((EK,), jnp.float32),
        pltpu.MemorySpace.VMEM_SHARED((NP, DP), jnp.float32),
        pltpu.SemaphoreType.DMA,
        pltpu.SemaphoreType.DMA,
        pltpu.SemaphoreType.DMA,
        pltpu.SemaphoreType.DMA,
    ],
    compiler_params=_sc_params,
)


EB = 2000


def _pad_emb_body(emb_ref, out_ref):
    out_ref[...] = jnp.concatenate(
        [emb_ref[...], jnp.zeros((EB, DP - D), jnp.float32)], axis=1)


def _pad_emb(emb):
    return pl.pallas_call(
        _pad_emb_body,
        grid=(NE // EB,),
        in_specs=[pl.BlockSpec((EB, D), lambda i: (i, 0))],
        out_specs=pl.BlockSpec((EB, DP), lambda i: (i, 0)),
        out_shape=jax.ShapeDtypeStruct((NE, DP), jnp.float32),
    )(emb)


def _wbuild_body(comp_ref, basis_ref, root_ref, w_out):
    basis2d = jnp.pad(basis_ref[...],
                      ((0, 0), (0, DP - D), (0, DP - D))).reshape(NB, DP * DP)
    w = lax.dot_general(comp_ref[...], basis2d, (((1,), (0,)), ((), ())),
                        preferred_element_type=jnp.float32)
    rootp = jnp.pad(root_ref[...], ((0, DP - D), (0, DP - D))).reshape(1, DP * DP)
    w_out[...] = jnp.concatenate([w, rootp], axis=0).reshape(TR, DP, DP)


def _wbuild(comp, basis, root):
    return pl.pallas_call(
        _wbuild_body,
        out_shape=jax.ShapeDtypeStruct((TR, DP, DP), jnp.float32),
    )(comp, basis, root)


BN = 1024


def _xw_body(x_ref, w_ref, out_ref):
    x = x_ref[...]
    for r in range(TR):
        out_ref[r] = lax.dot_general(
            x, w_ref[r], (((1,), (0,)), ((), ())),
            preferred_element_type=jnp.float32)


def _xw_matmul(x, w):
    return pl.pallas_call(
        _xw_body,
        grid=(NP // BN,),
        in_specs=[
            pl.BlockSpec((BN, DP), lambda i: (i, 0)),
            pl.BlockSpec((TR, DP, DP), lambda i: (0, 0, 0)),
        ],
        out_specs=pl.BlockSpec((TR, BN, DP), lambda i: (0, i, 0)),
        out_shape=jax.ShapeDtypeStruct((TR, NP, DP), jnp.float32),
    )(x, w)


def _combine_matmul_body(agg_ref, self_ref, bias_ref, w_ref, out_ref):
    h = agg_ref[0] + agg_ref[1] + self_ref[...] + bias_ref[...]
    h = jnp.maximum(h, 0.0)
    for r in range(TR):
        out_ref[r] = lax.dot_general(
            h, w_ref[r], (((1,), (0,)), ((), ())),
            preferred_element_type=jnp.float32)


def _combine_matmul(agg, selfloop, bias_p, w):
    return pl.pallas_call(
        _combine_matmul_body,
        grid=(NP // BN,),
        in_specs=[
            pl.BlockSpec((NC, BN, DP), lambda i: (0, i, 0)),
            pl.BlockSpec((BN, DP), lambda i: (i, 0)),
            pl.BlockSpec((1, DP), lambda i: (0, 0)),
            pl.BlockSpec((TR, DP, DP), lambda i: (0, 0, 0)),
        ],
        out_specs=pl.BlockSpec((TR, BN, DP), lambda i: (0, i, 0)),
        out_shape=jax.ShapeDtypeStruct((TR, NP, DP), jnp.float32),
    )(agg, selfloop, bias_p, w)


def _combine_final_body(agg_ref, self_ref, bias_ref, out_ref):
    out_ref[...] = agg_ref[0] + agg_ref[1] + self_ref[...] + bias_ref[...]


def _combine_final(agg, selfloop, bias_p):
    return pl.pallas_call(
        _combine_final_body,
        grid=(NP // BN,),
        in_specs=[
            pl.BlockSpec((NC, BN, DP), lambda i: (0, i, 0)),
            pl.BlockSpec((BN, DP), lambda i: (i, 0)),
            pl.BlockSpec((1, DP), lambda i: (0, 0)),
        ],
        out_specs=pl.BlockSpec((BN, DP), lambda i: (i, 0)),
        out_shape=jax.ShapeDtypeStruct((NP, DP), jnp.float32),
    )(agg, selfloop, bias_p)


def kernel(entity, edge_index, edge_type, emb_table,
           comp1, basis1, root1, bias1, comp2, basis2, root2, bias2):
    entity = entity.astype(jnp.int32)
    edge_index = edge_index.astype(jnp.int32)
    edge_type = edge_type.astype(jnp.int32)

    pad_e = EP - E
    src_p = jnp.concatenate([edge_index[0], jnp.zeros((pad_e,), jnp.int32)])
    dst_p = jnp.concatenate([edge_index[1],
                             jnp.full((pad_e,), DUMP, jnp.int32)])
    t_p = jnp.concatenate([edge_type, jnp.zeros((pad_e,), jnp.int32)])
    ent_p = jnp.concatenate([entity, jnp.zeros((NP - N,), jnp.int32)])
    zeros_blk = jnp.zeros((AGG_W, DP), jnp.float32)
    zeros_cnt = jnp.zeros((CNT_W,), jnp.float32)
    bias1_p = jnp.concatenate([bias1, jnp.zeros((DP - D,), jnp.float32)])
    bias1_p = bias1_p.reshape(1, DP)
    bias2_p = jnp.concatenate([bias2, jnp.zeros((DP - D,), jnp.float32)])
    bias2_p = bias2_p.reshape(1, DP)

    emb_p = _pad_emb(emb_table)
    x0 = _sc_lookup(ent_p, emb_p)
    cnt_part = _sc_counts(dst_p, t_p, zeros_cnt)
    invcnt = _sc_inv(cnt_part)

    w1 = _wbuild(comp1, basis1, root1)
    xw1 = _xw_matmul(x0, w1)
    xw1_rows = xw1.reshape(TR * NP, DP)
    agg1 = _sc_edge_agg(xw1_rows, invcnt, zeros_blk, src_p, dst_p, t_p)

    w2 = _wbuild(comp2, basis2, root2)
    xw2 = _combine_matmul(agg1, xw1[R], bias1_p, w2)
    xw2_rows = xw2.reshape(TR * NP, DP)
    agg2 = _sc_edge_agg(xw2_rows, invcnt, zeros_blk, src_p, dst_p, t_p)

    out = _combine_final(agg2, xw2[R], bias2_p)
    return out[:N, :D]


# speedup vs baseline: 3.6734x; 1.0156x over previous
import jax
import jax.numpy as jnp
from jax import lax
from jax.experimental import pallas as pl
from jax.experimental.pallas import tpu as pltpu
from jax.experimental.pallas import tpu_sc as plsc

N = 10000
E = 160000
D = 100
R = 20
NB = 4
NE = 100000

NC = 2
NS = 16
NW = NC * NS

NP = 10240
ROWS_W = NP // NW
LCH = 64
EP = 163840
EW = EP // NW
EK = 64
NCHUNK = EW // EK
NRP = 200704
SEGW = NRP // NW
CNT_W = NRP // NS
DP = 128
TR = R + 1
DUMP = N
NAGG = 10048
AGG_W = 632
AGG_LAST = NAGG - (NS - 1) * AGG_W

_mesh = plsc.VectorSubcoreMesh(core_axis_name="c", subcore_axis_name="s")
_sc_params = pltpu.CompilerParams(needs_layout_passes=False)


def _sc_lookup_body(entity_ref, emb_ref, x_out, idx_v, rows_v, sem):
    c = lax.axis_index("c")
    s = lax.axis_index("s")
    w = s * NC + c
    pltpu.sync_copy(entity_ref.at[pl.ds(w * ROWS_W, ROWS_W)], idx_v)
    cps = [
        pltpu.async_copy(emb_ref.at[idx_v.at[pl.ds(k * LCH, LCH)]],
                         rows_v.at[pl.ds(k * LCH, LCH)], sem)
        for k in range(ROWS_W // LCH)
    ]
    for cp in cps:
        cp.wait()
    pltpu.sync_copy(rows_v, x_out.at[pl.ds(w * ROWS_W, ROWS_W)])


_sc_lookup = pl.kernel(
    _sc_lookup_body,
    out_type=jax.ShapeDtypeStruct((NP, DP), jnp.float32),
    mesh=_mesh,
    scratch_types=[
        pltpu.VMEM((ROWS_W,), jnp.int32),
        pltpu.VMEM((ROWS_W, DP), jnp.float32),
        pltpu.SemaphoreType.DMA,
    ],
    compiler_params=_sc_params,
)


def _sc_counts_body(dst_ref, t_ref, zeros_ref, part_out,
                    ebuf_d, ebuf_t, segbuf, ones_v, cnt_sh, sem):
    c = lax.axis_index("c")
    s = lax.axis_index("s")
    w = s * NC + c

    pltpu.sync_copy(zeros_ref, cnt_sh.at[pl.ds(s * CNT_W, CNT_W)])

    def ones_body(i, _):
        ones_v[pl.ds(i * 16, 16)] = jnp.ones((16,), jnp.float32)
        return 0
    lax.fori_loop(0, EK // 16, ones_body, 0, unroll=8)

    pltpu.sync_copy(dst_ref.at[pl.ds(w * EW, EW)], ebuf_d)
    pltpu.sync_copy(t_ref.at[pl.ds(w * EW, EW)], ebuf_t)
    plsc.subcore_barrier()

    def chunk_body(ci, _):
        def bidx(vi, _):
            d16 = ebuf_d[pl.ds(ci * EK + vi * 16, 16)]
            t16 = ebuf_t[pl.ds(ci * EK + vi * 16, 16)]
            segbuf[ci, pl.ds(vi * 16, 16)] = d16 * R + t16
            return 0
        lax.fori_loop(0, EK // 16, bidx, 0, unroll=8)
        pltpu.async_copy(ones_v, cnt_sh.at[segbuf.at[ci]], sem, add=True)
        return 0
    lax.fori_loop(0, NCHUNK, chunk_body, 0)

    def drain_body(ci, _):
        pltpu.make_async_copy(ones_v, cnt_sh.at[segbuf.at[ci]], sem).wait()
        return 0
    lax.fori_loop(0, NCHUNK, drain_body, 0)

    plsc.subcore_barrier()
    pltpu.sync_copy(cnt_sh.at[pl.ds(s * CNT_W, CNT_W)],
                    part_out.at[c, pl.ds(s * CNT_W, CNT_W)])


_sc_counts = pl.kernel(
    _sc_counts_body,
    out_type=jax.ShapeDtypeStruct((NC, NRP), jnp.float32),
    mesh=_mesh,
    scratch_types=[
        pltpu.VMEM((EW,), jnp.int32),
        pltpu.VMEM((EW,), jnp.int32),
        pltpu.VMEM((NCHUNK, EK), jnp.int32),
        pltpu.VMEM((EK,), jnp.float32),
        pltpu.MemorySpace.VMEM_SHARED((NRP,), jnp.float32),
        pltpu.SemaphoreType.DMA,
    ],
    compiler_params=_sc_params,
)


def _sc_inv_body(part_ref, inv_out, buf_a, buf_b):
    c = lax.axis_index("c")
    s = lax.axis_index("s")
    w = s * NC + c
    lo = w * SEGW
    pltpu.sync_copy(part_ref.at[0, pl.ds(lo, SEGW)], buf_a)
    pltpu.sync_copy(part_ref.at[1, pl.ds(lo, SEGW)], buf_b)

    def inv_body(i, _):
        v = buf_a[pl.ds(i * 16, 16)] + buf_b[pl.ds(i * 16, 16)]
        buf_a[pl.ds(i * 16, 16)] = 1.0 / jnp.maximum(v, 1.0)
        return 0
    lax.fori_loop(0, SEGW // 16, inv_body, 0, unroll=8)
    pltpu.sync_copy(buf_a, inv_out.at[pl.ds(lo, SEGW)])


_sc_inv = pl.kernel(
    _sc_inv_body,
    out_type=jax.ShapeDtypeStruct((NRP,), jnp.float32),
    mesh=_mesh,
    scratch_types=[
        pltpu.VMEM((SEGW,), jnp.float32),
        pltpu.VMEM((SEGW,), jnp.float32),
    ],
    compiler_params=_sc_params,
)


NBUF = 4


def _sc_edge_agg_body(xw_ref, inv_ref, zeros_ref, src_ref, dst_ref, t_ref,
                      out_ref,
                      ebuf_s, ebuf_d, ebuf_t,
                      gidx0, sidx0, didx0, rows0, sv0,
                      gidx1, sidx1, didx1, rows1, sv1,
                      gidx2, sidx2, didx2, rows2, sv2,
                      gidx3, sidx3, didx3, rows3, sv3,
                      agg_sh,
                      semg0, semg1, semg2, semg3,
                      sems0, sems1, sems2, sems3):
    c = lax.axis_index("c")
    s = lax.axis_index("s")
    w = s * NC + c
    gidx = [gidx0, gidx1, gidx2, gidx3]
    sidx = [sidx0, sidx1, sidx2, sidx3]
    didx = [didx0, didx1, didx2, didx3]
    rows = [rows0, rows1, rows2, rows3]
    sv = [sv0, sv1, sv2, sv3]
    semg = [semg0, semg1, semg2, semg3]
    sems = [sems0, sems1, sems2, sems3]

    @pl.when(s < NS - 1)
    def _():
        pltpu.sync_copy(zeros_ref, agg_sh.at[pl.ds(s * AGG_W, AGG_W)])

    @pl.when(s == NS - 1)
    def _():
        pltpu.sync_copy(zeros_ref.at[pl.ds(0, AGG_LAST)],
                        agg_sh.at[pl.ds((NS - 1) * AGG_W, AGG_LAST)])
    plsc.subcore_barrier()

    pltpu.sync_copy(src_ref.at[pl.ds(w * EW, EW)], ebuf_s)
    pltpu.sync_copy(dst_ref.at[pl.ds(w * EW, EW)], ebuf_d)
    pltpu.sync_copy(t_ref.at[pl.ds(w * EW, EW)], ebuf_t)

    def build(ci, b):
        def bidx(vi, _):
            s16 = ebuf_s[pl.ds(ci * EK + vi * 16, 16)]
            d16 = ebuf_d[pl.ds(ci * EK + vi * 16, 16)]
            t16 = ebuf_t[pl.ds(ci * EK + vi * 16, 16)]
            gidx[b][pl.ds(vi * 16, 16)] = t16 * NP + s16
            sidx[b][pl.ds(vi * 16, 16)] = d16 * R + t16
            didx[b][pl.ds(vi * 16, 16)] = d16
            return 0
        lax.fori_loop(0, EK // 16, bidx, 0, unroll=8)

    def fire(b):
        pltpu.async_copy(xw_ref.at[gidx[b]], rows[b], semg[b])
        pltpu.async_copy(inv_ref.at[sidx[b]], sv[b], semg[b])

    def wait_gather(b):
        pltpu.make_async_copy(xw_ref.at[gidx[b]], rows[b], semg[b]).wait()
        pltpu.make_async_copy(inv_ref.at[sidx[b]], sv[b], semg[b]).wait()

    def scale(b):
        def scale_row(r, _):
            sc = plsc.load_gather(sv[b], [jnp.full((16,), r, jnp.int32)])
            for g in range(DP // 16):
                v = rows[b][r, pl.ds(g * 16, 16)]
                rows[b][r, pl.ds(g * 16, 16)] = v * sc
            return 0
        lax.fori_loop(0, EK, scale_row, 0, unroll=4)

    def fire_scatter(b):
        pltpu.async_copy(rows[b], agg_sh.at[didx[b]], sems[b], add=True)

    def wait_scatter(b):
        pltpu.make_async_copy(rows[b], agg_sh.at[didx[b]], sems[b]).wait()

    for j in range(NBUF - 1):
        build(j, j)
        fire(j)

    def quad_body(ci, _):
        for j in range(NBUF):
            ch = NBUF * ci + j
            nxt = ch + NBUF - 1
            k = (j + NBUF - 1) % NBUF
            wait_gather(j)
            scale(j)
            fire_scatter(j)

            @pl.when((nxt < NCHUNK) & (ch >= 1))
            def _():
                wait_scatter(k)

            @pl.when(nxt < NCHUNK)
            def _():
                build(nxt, k)
                fire(k)
        return 0
    lax.fori_loop(0, NCHUNK // NBUF, quad_body, 0)
    for j in range(NBUF):
        wait_scatter(j)

    plsc.subcore_barrier()

    @pl.when(s < NS - 1)
    def _():
        pltpu.sync_copy(agg_sh.at[pl.ds(s * AGG_W, AGG_W)],
                        out_ref.at[c, pl.ds(s * AGG_W, AGG_W)])

    @pl.when(s == NS - 1)
    def _():
        pltpu.sync_copy(agg_sh.at[pl.ds((NS - 1) * AGG_W, AGG_LAST)],
                        out_ref.at[c, pl.ds((NS - 1) * AGG_W, AGG_LAST)])


_sc_edge_agg = pl.kernel(
    _sc_edge_agg_body,
    out_type=jax.ShapeDtypeStruct((NC, NP, DP), jnp.float32),
    mesh=_mesh,
    scratch_types=(
        [pltpu.VMEM((EW,), jnp.int32)] * 3
        + [pltpu.VMEM((EK,), jnp.int32),
           pltpu.VMEM((EK,), jnp.int32),
           pltpu.VMEM((EK,), jnp.int32),
           pltpu.VMEM((EK, DP), jnp.float32),
           pltpu.VMEM((EK,), jnp.float32)] * NBUF
        + [pltpu.MemorySpace.VMEM_SHARED((NAGG, DP), jnp.float32)]
        + [pltpu.SemaphoreType.DMA] * (2 * NBUF)
    ),
    compiler_params=_sc_params,
)


EB = 2000


def _pad_emb_body(emb_ref, out_ref):
    out_ref[...] = jnp.concatenate(
        [emb_ref[...], jnp.zeros((EB, DP - D), jnp.float32)], axis=1)


def _pad_emb(emb):
    return pl.pallas_call(
        _pad_emb_body,
        grid=(NE // EB,),
        in_specs=[pl.BlockSpec((EB, D), lambda i: (i, 0))],
        out_specs=pl.BlockSpec((EB, DP), lambda i: (i, 0)),
        out_shape=jax.ShapeDtypeStruct((NE, DP), jnp.float32),
    )(emb)


def _wbuild_body(comp_ref, basis_ref, root_ref, w_out):
    basis2d = jnp.pad(basis_ref[...],
                      ((0, 0), (0, DP - D), (0, DP - D))).reshape(NB, DP * DP)
    w = lax.dot_general(comp_ref[...], basis2d, (((1,), (0,)), ((), ())),
                        preferred_element_type=jnp.float32)
    rootp = jnp.pad(root_ref[...], ((0, DP - D), (0, DP - D))).reshape(1, DP * DP)
    w_out[...] = jnp.concatenate([w, rootp], axis=0).reshape(TR, DP, DP)


def _wbuild(comp, basis, root):
    return pl.pallas_call(
        _wbuild_body,
        out_shape=jax.ShapeDtypeStruct((TR, DP, DP), jnp.float32),
    )(comp, basis, root)


BN = 1024


def _xw_body(x_ref, w_ref, out_ref):
    x = x_ref[...]
    for r in range(TR):
        out_ref[r] = lax.dot_general(
            x, w_ref[r], (((1,), (0,)), ((), ())),
            preferred_element_type=jnp.float32)


def _xw_matmul(x, w):
    return pl.pallas_call(
        _xw_body,
        grid=(NP // BN,),
        in_specs=[
            pl.BlockSpec((BN, DP), lambda i: (i, 0)),
            pl.BlockSpec((TR, DP, DP), lambda i: (0, 0, 0)),
        ],
        out_specs=pl.BlockSpec((TR, BN, DP), lambda i: (0, i, 0)),
        out_shape=jax.ShapeDtypeStruct((TR, NP, DP), jnp.float32),
    )(x, w)


def _combine_matmul_body(agg_ref, self_ref, bias_ref, w_ref, out_ref):
    h = agg_ref[0] + agg_ref[1] + self_ref[...] + bias_ref[...]
    h = jnp.maximum(h, 0.0)
    for r in range(TR):
        out_ref[r] = lax.dot_general(
            h, w_ref[r], (((1,), (0,)), ((), ())),
            preferred_element_type=jnp.float32)


def _combine_matmul(agg, selfloop, bias_p, w):
    return pl.pallas_call(
        _combine_matmul_body,
        grid=(NP // BN,),
        in_specs=[
            pl.BlockSpec((NC, BN, DP), lambda i: (0, i, 0)),
            pl.BlockSpec((BN, DP), lambda i: (i, 0)),
            pl.BlockSpec((1, DP), lambda i: (0, 0)),
            pl.BlockSpec((TR, DP, DP), lambda i: (0, 0, 0)),
        ],
        out_specs=pl.BlockSpec((TR, BN, DP), lambda i: (0, i, 0)),
        out_shape=jax.ShapeDtypeStruct((TR, NP, DP), jnp.float32),
    )(agg, selfloop, bias_p, w)


def _combine_final_body(agg_ref, self_ref, bias_ref, out_ref):
    out_ref[...] = agg_ref[0] + agg_ref[1] + self_ref[...] + bias_ref[...]


def _combine_final(agg, selfloop, bias_p):
    return pl.pallas_call(
        _combine_final_body,
        grid=(NP // BN,),
        in_specs=[
            pl.BlockSpec((NC, BN, DP), lambda i: (0, i, 0)),
            pl.BlockSpec((BN, DP), lambda i: (i, 0)),
            pl.BlockSpec((1, DP), lambda i: (0, 0)),
        ],
        out_specs=pl.BlockSpec((BN, DP), lambda i: (i, 0)),
        out_shape=jax.ShapeDtypeStruct((NP, DP), jnp.float32),
    )(agg, selfloop, bias_p)


def kernel(entity, edge_index, edge_type, emb_table,
           comp1, basis1, root1, bias1, comp2, basis2, root2, bias2):
    entity = entity.astype(jnp.int32)
    edge_index = edge_index.astype(jnp.int32)
    edge_type = edge_type.astype(jnp.int32)

    pad_e = EP - E
    src_p = jnp.concatenate([edge_index[0], jnp.zeros((pad_e,), jnp.int32)])
    dst_p = jnp.concatenate([edge_index[1],
                             jnp.full((pad_e,), DUMP, jnp.int32)])
    t_p = jnp.concatenate([edge_type, jnp.zeros((pad_e,), jnp.int32)])
    ent_p = jnp.concatenate([entity, jnp.zeros((NP - N,), jnp.int32)])
    zeros_blk = jnp.zeros((AGG_W, DP), jnp.float32)
    zeros_cnt = jnp.zeros((CNT_W,), jnp.float32)
    bias1_p = jnp.concatenate([bias1, jnp.zeros((DP - D,), jnp.float32)])
    bias1_p = bias1_p.reshape(1, DP)
    bias2_p = jnp.concatenate([bias2, jnp.zeros((DP - D,), jnp.float32)])
    bias2_p = bias2_p.reshape(1, DP)

    emb_p = _pad_emb(emb_table)
    x0 = _sc_lookup(ent_p, emb_p)
    cnt_part = _sc_counts(dst_p, t_p, zeros_cnt)
    invcnt = _sc_inv(cnt_part)

    w1 = _wbuild(comp1, basis1, root1)
    xw1 = _xw_matmul(x0, w1)
    xw1_rows = xw1.reshape(TR * NP, DP)
    agg1 = _sc_edge_agg(xw1_rows, invcnt, zeros_blk, src_p, dst_p, t_p)

    w2 = _wbuild(comp2, basis2, root2)
    xw2 = _combine_matmul(agg1, xw1[R], bias1_p, w2)
    xw2_rows = xw2.reshape(TR * NP, DP)
    agg2 = _sc_edge_agg(xw2_rows, invcnt, zeros_blk, src_p, dst_p, t_p)

    out = _combine_final(agg2, xw2[R], bias2_p)
    return out[:N, :D]
